# Initial kernel scaffold; baseline (speedup 1.0000x reference)
import jax
import jax.numpy as jnp
from jax.experimental import pallas as pl


def kernel(h, r, pos_t, neg_t, entity_user_embed, relation_embed, h_trans_w1, h_trans_w2, h_bias_b, r_trans_w1, r_trans_w2, r_bias_b, sem_trans_w):
    raise NotImplementedError("write your pallas kernel here")


# speedup vs baseline: 4.5083x; 4.5083x over previous
import functools

import jax
import jax.numpy as jnp
from jax import lax
from jax.experimental import pallas as pl
from jax.experimental.pallas import tpu as pltpu
from jax.experimental.pallas import tpu_sc as plsc

BATCH = 4096
EMBED_DIM = 64
KG_LAMBDA = 1e-05

_NC, _NS = 2, 16
_NW = _NC * _NS
_BPW = BATCH // _NW


def _sc_gather(h, r, pos_t, neg_t, table, rel_table):
    row = jax.ShapeDtypeStruct((BATCH, EMBED_DIM), jnp.float32)

    @functools.partial(
        pl.kernel,
        mesh=plsc.VectorSubcoreMesh(core_axis_name="c", subcore_axis_name="s"),
        out_type=[row, row, row, row],
        scratch_types=[
            pltpu.VMEM((_BPW,), jnp.int32),
            pltpu.VMEM((_BPW, EMBED_DIM), jnp.float32),
            pltpu.SemaphoreType.DMA,
        ],
        compiler_params=pltpu.CompilerParams(use_tc_tiling_on_sc=False),
    )
    def k(h_hbm, r_hbm, pos_hbm, neg_hbm, tab_hbm, rel_hbm,
          out_h, out_r, out_pos, out_neg, idx_v, rows_v, sem):
        wid = lax.axis_index("s") * _NC + lax.axis_index("c")
        base = wid * _BPW
        for idx_hbm, tbl, out in ((h_hbm, tab_hbm, out_h),
                                  (r_hbm, rel_hbm, out_r),
                                  (pos_hbm, tab_hbm, out_pos),
                                  (neg_hbm, tab_hbm, out_neg)):
            pltpu.sync_copy(idx_hbm.at[pl.ds(base, _BPW)], idx_v)
            pltpu.async_copy(tbl.at[idx_v], rows_v, sem).wait()
            pltpu.sync_copy(rows_v, out.at[pl.ds(base, _BPW)])

    return k(h, r, pos_t, neg_t, table, rel_table)


def _tc_body(h_ref, r_ref, pos_ref, neg_ref, hw1_ref, hw2_ref, hb_ref,
             rw1_ref, rw2_ref, rb_ref, w_ref, out_ref):
    he = h_ref[...]
    re = r_ref[...]
    a1 = jnp.sum(re * hw1_ref[...], axis=1, keepdims=True)
    a2 = jnp.sum(he * rw2_ref[...], axis=1, keepdims=True)
    b1 = jnp.sum(re * hw2_ref[...], axis=1, keepdims=True)
    b2 = jnp.sum(he * rw1_ref[...], axis=1, keepdims=True)
    cross_h = he * a1 + re * a2 + hb_ref[...]
    cross_r = he * b1 + re * b2 + rb_ref[...]
    w1 = w_ref[:EMBED_DIM, :]
    w2 = w_ref[EMBED_DIM:, :]
    pred = (jnp.dot(cross_h, w1, preferred_element_type=jnp.float32)
            + jnp.dot(cross_r, w2, preferred_element_type=jnp.float32))
    pos = pos_ref[...]
    neg = neg_ref[...]
    x = jnp.sum(pred * (pos - neg), axis=1, keepdims=True)
    nls = jnp.maximum(-x, 0.0) + jnp.log1p(jnp.exp(-jnp.abs(x)))
    kg_loss = jnp.sum(nls) / BATCH
    l2 = (jnp.sum(cross_h * cross_h) + jnp.sum(cross_r * cross_r)
          + jnp.sum(pos * pos) + jnp.sum(neg * neg)) / (2.0 * BATCH)
    out_ref[0, 0] = kg_loss + KG_LAMBDA * l2


def kernel(h, r, pos_t, neg_t, entity_user_embed, relation_embed,
           h_trans_w1, h_trans_w2, h_bias_b, r_trans_w1, r_trans_w2, r_bias_b,
           sem_trans_w):
    h_e, r_e, pos_e, neg_e = _sc_gather(
        h.astype(jnp.int32), r.astype(jnp.int32), pos_t.astype(jnp.int32),
        neg_t.astype(jnp.int32), entity_user_embed, relation_embed)

    out = pl.pallas_call(
        _tc_body,
        out_shape=jax.ShapeDtypeStruct((1, 1), jnp.float32),
        in_specs=[pl.BlockSpec(memory_space=pltpu.VMEM)] * 11,
        out_specs=pl.BlockSpec(memory_space=pltpu.SMEM),
    )(h_e, r_e, pos_e, neg_e,
      h_trans_w1.reshape(1, EMBED_DIM), h_trans_w2.reshape(1, EMBED_DIM),
      h_bias_b.reshape(1, EMBED_DIM),
      r_trans_w1.reshape(1, EMBED_DIM), r_trans_w2.reshape(1, EMBED_DIM),
      r_bias_b.reshape(1, EMBED_DIM), sem_trans_w)
    return out[0, 0]
